# Initial kernel scaffold; baseline (speedup 1.0000x reference)
import jax
import jax.numpy as jnp
from jax.experimental import pallas as pl


def kernel(sensory_input, context, buffer, w_light, mem, W_q, W_o, write_ptr):
    raise NotImplementedError("write your pallas kernel here")


# speedup vs baseline: 1.9764x; 1.9764x over previous
import functools

import jax
import jax.numpy as jnp
from jax.experimental import pallas as pl
from jax.experimental.pallas import tpu as pltpu

_D = 256
_INV_SQRT_D = 1.0 / 16.0


def _stats_kernel(x_ref, w_ref, score_ref, pooled_ref, *, nsteps, total):
    i = pl.program_id(0)
    x = x_ref[...]
    dot = jnp.dot(x, w_ref[...].T,
                  preferred_element_type=jnp.float32)
    s = jnp.sum(jnp.abs(dot)).reshape(1, 1)
    p = jnp.sum(x, axis=0, keepdims=True)

    @pl.when(i == 0)
    def _():
        score_ref[...] = s
        pooled_ref[...] = p

    @pl.when(i > 0)
    def _():
        score_ref[...] += s
        pooled_ref[...] += p

    @pl.when(i == nsteps - 1)
    def _():
        inv = 1.0 / total
        score_ref[...] = score_ref[...] * inv
        pooled_ref[...] = pooled_ref[...] * inv


def _main_kernel(wp_ref, x_ref, score_ref, pooled_ref, buffer_ref,
                 mem_ref, wq_ref, wo_ref, out_ref):
    wp = wp_ref[0]
    row = jax.lax.broadcasted_iota(jnp.int32, (5, 1), 0)
    buf = jnp.where(row == wp, pooled_ref[...], buffer_ref[...])

    score = score_ref[...]
    fire = jax.nn.sigmoid(score - 2.0)
    temp = jnp.maximum(0.5, 1.0 - 0.3 * fire)

    x = x_ref[...]
    logits = jnp.dot(x, buf.T, preferred_element_type=jnp.float32)
    logits = logits * _INV_SQRT_D
    m = jnp.max(logits, axis=-1, keepdims=True)
    e = jnp.exp(logits - m)
    attn = e / jnp.sum(e, axis=-1, keepdims=True)
    filtered = x + jnp.dot(attn, buf, preferred_element_type=jnp.float32)

    q = jnp.dot(filtered, wq_ref[...], preferred_element_type=jnp.float32)
    wl = jnp.dot(q, mem_ref[...].T, preferred_element_type=jnp.float32)
    wl = wl * (_INV_SQRT_D / temp)
    m2 = jnp.max(wl, axis=-1, keepdims=True)
    e2 = jnp.exp(wl - m2)
    wa = e2 / jnp.sum(e2, axis=-1, keepdims=True)
    read = jnp.dot(wa, mem_ref[...], preferred_element_type=jnp.float32)
    out_ref[...] = jnp.dot(read, wo_ref[...],
                           preferred_element_type=jnp.float32) + filtered


@jax.jit
def kernel(sensory_input, context, buffer, w_light, mem, W_q, W_o, write_ptr):
    B, S, d = sensory_input.shape
    total = B * S
    x = sensory_input.reshape(total, d)
    w2 = w_light.reshape(1, d)
    wp = jnp.asarray(write_ptr, dtype=jnp.int32).reshape(1)

    R1 = 4096
    n1 = total // R1
    score, pooled = pl.pallas_call(
        functools.partial(_stats_kernel, nsteps=n1, total=float(total)),
        grid=(n1,),
        in_specs=[
            pl.BlockSpec((R1, d), lambda i: (i, 0)),
            pl.BlockSpec((1, d), lambda i: (0, 0)),
        ],
        out_specs=[
            pl.BlockSpec((1, 1), lambda i: (0, 0)),
            pl.BlockSpec((1, d), lambda i: (0, 0)),
        ],
        out_shape=[
            jax.ShapeDtypeStruct((1, 1), jnp.float32),
            jax.ShapeDtypeStruct((1, d), jnp.float32),
        ],
    )(x, w2)

    R2 = 2048
    n2 = total // R2
    out = pl.pallas_call(
        _main_kernel,
        grid=(n2,),
        in_specs=[
            pl.BlockSpec(memory_space=pltpu.SMEM),
            pl.BlockSpec((R2, d), lambda i: (i, 0)),
            pl.BlockSpec((1, 1), lambda i: (0, 0)),
            pl.BlockSpec((1, d), lambda i: (0, 0)),
            pl.BlockSpec((5, d), lambda i: (0, 0)),
            pl.BlockSpec((7, d), lambda i: (0, 0)),
            pl.BlockSpec((d, d), lambda i: (0, 0)),
            pl.BlockSpec((d, d), lambda i: (0, 0)),
        ],
        out_specs=pl.BlockSpec((R2, d), lambda i: (i, 0)),
        out_shape=jax.ShapeDtypeStruct((total, d), jnp.float32),
    )(wp, x, score, pooled, buffer, mem, W_q, W_o)

    return out.reshape(B, S, d)
